# Initial kernel scaffold; baseline (speedup 1.0000x reference)
import jax
import jax.numpy as jnp
from jax.experimental import pallas as pl


def kernel(x, sample_index, mask_index, W, b):
    raise NotImplementedError("write your pallas kernel here")


# speedup vs baseline: 1.8243x; 1.8243x over previous
import functools

import jax
import jax.numpy as jnp
from jax import lax
from jax.experimental import pallas as pl
from jax.experimental.pallas import tpu as pltpu
from jax.experimental.pallas import tpu_sc as plsc

B = 64
PATCH = 16
IN_CHANS = 3
EMBED = 768
NUM_PATCHES = 1024
N_VIS = 256
N_MASK = 768
ROWS = 1 + NUM_PATCHES

NC = 2
NS = 16
NW = NC * NS
HALF = B // 2

GRAY = 127.0 / 255.0


def _body(x_hbm, samp_hbm, mask_hbm, w_hbm, b_hbm, out_hbm,
          samp_v, mask_v, code_s, w0_v, b_v, constbuf, zerobuf, xbuf):
    wid = lax.axis_index("s") * NC + lax.axis_index("c")

    pltpu.sync_copy(samp_hbm, samp_v)
    pltpu.sync_copy(mask_hbm, mask_v)
    pltpu.sync_copy(w_hbm.at[pl.ds(0, 1), :], w0_v)
    pltpu.sync_copy(b_hbm.at[pl.ds(0, 16)], b_v)

    def sum_step(i, acc):
        return acc + w0_v[0, pl.ds(i * 16, 16)]
    acc = lax.fori_loop(0, EMBED // 16, sum_step, jnp.zeros((16,), jnp.float32))
    tot = acc[0]
    for i in range(1, 16):
        tot = tot + acc[i]
    s = tot * jnp.float32(GRAY) + b_v[...][0]
    vs = jnp.full((16,), s, dtype=jnp.float32)
    vz = jnp.zeros((16,), jnp.float32)

    def init_step(t, _):
        code_s[t] = jnp.int32(-1)
        return 0
    lax.fori_loop(0, ROWS, init_step, 0)

    code_s[0] = jnp.int32(0)

    def samp_step(g, _):
        v = samp_v[pl.ds(g * 16, 16)]
        for i in range(16):
            code_s[v[i]] = g * 16 + i + 1
        return 0
    lax.fori_loop(0, N_VIS // 16, samp_step, 0)

    def mask_step(g, _):
        v = mask_v[pl.ds(g * 16, 16)]
        for i in range(16):
            code_s[v[i]] = jnp.int32(-2)
        return 0
    lax.fori_loop(0, N_MASK // 16, mask_step, 0)

    def fill_row(row, _):
        def fill_col(col, _):
            constbuf[row, 0, pl.ds(col * 16, 16)] = vs
            zerobuf[row, 0, pl.ds(col * 16, 16)] = vz
            return 0
        lax.fori_loop(0, EMBED // 16, fill_col, 0)
        return 0
    lax.fori_loop(0, HALF, fill_row, 0)

    def do_row(r):
        c = code_s[r]
        for b0 in (0, HALF):
            @pl.when(c == -2)
            def _():
                pltpu.sync_copy(constbuf,
                                out_hbm.at[pl.ds(b0, HALF), pl.ds(r, 1), :])

            @pl.when(c == -1)
            def _():
                pltpu.sync_copy(zerobuf,
                                out_hbm.at[pl.ds(b0, HALF), pl.ds(r, 1), :])

            @pl.when(c >= 0)
            def _():
                pltpu.sync_copy(x_hbm.at[pl.ds(b0, HALF), pl.ds(c, 1), :],
                                xbuf)
                pltpu.sync_copy(xbuf,
                                out_hbm.at[pl.ds(b0, HALF), pl.ds(r, 1), :])

    def row_step(k, _):
        do_row(k * NW + wid)
        return 0
    lax.fori_loop(0, NUM_PATCHES // NW, row_step, 0)

    @pl.when(wid == 0)
    def _():
        do_row(NUM_PATCHES)


@functools.partial(jax.jit, static_argnames=())
def kernel(x, sample_index, mask_index, W, b):
    w2d = W.reshape(EMBED, IN_CHANS * PATCH * PATCH)
    run = pl.kernel(
        _body,
        mesh=plsc.VectorSubcoreMesh(core_axis_name="c", subcore_axis_name="s"),
        out_type=jax.ShapeDtypeStruct((B, ROWS, EMBED), jnp.float32),
        scratch_types=[
            pltpu.VMEM((N_VIS,), jnp.int32),
            pltpu.VMEM((N_MASK,), jnp.int32),
            pltpu.SMEM((ROWS,), jnp.int32),
            pltpu.VMEM((1, EMBED), jnp.float32),
            pltpu.VMEM((16,), jnp.float32),
            pltpu.VMEM((HALF, 1, EMBED), jnp.float32),
            pltpu.VMEM((HALF, 1, EMBED), jnp.float32),
            pltpu.VMEM((HALF, 1, EMBED), jnp.float32),
        ],
    )
    return run(x, sample_index, mask_index, w2d, b)


# speedup vs baseline: 1.8914x; 1.0368x over previous
import functools

import jax
import jax.numpy as jnp
from jax import lax
from jax.experimental import pallas as pl
from jax.experimental.pallas import tpu as pltpu
from jax.experimental.pallas import tpu_sc as plsc

B = 64
PATCH = 16
IN_CHANS = 3
EMBED = 768
NUM_PATCHES = 1024
N_VIS = 256
N_MASK = 768
ROWS = 1 + NUM_PATCHES

NC = 2
NS = 16
NW = NC * NS
NBH = NW // 2
HALF = B // 2
LAG = 8

GRAY = 127.0 / 255.0


def _body(x_hbm, samp_hbm, mask_hbm, w_hbm, b_hbm, out_hbm,
          samp_v, mask_v, code_s, w0_v, b_v, constbuf, zerobuf, xbuf,
          sem_w, sem_x):
    wid = lax.axis_index("s") * NC + lax.axis_index("c")
    b0 = (wid // NBH) * HALF
    rgrp = wid % NBH

    pltpu.sync_copy(samp_hbm, samp_v)
    pltpu.sync_copy(mask_hbm, mask_v)
    pltpu.sync_copy(w_hbm.at[pl.ds(0, 1), :], w0_v)
    pltpu.sync_copy(b_hbm.at[pl.ds(0, 16)], b_v)

    def sum_step(i, acc):
        return acc + w0_v[0, pl.ds(i * 16, 16)]
    acc = lax.fori_loop(0, EMBED // 16, sum_step, jnp.zeros((16,), jnp.float32))
    tot = acc[0]
    for i in range(1, 16):
        tot = tot + acc[i]
    s = tot * jnp.float32(GRAY) + b_v[...][0]
    vs = jnp.full((16,), s, dtype=jnp.float32)
    vz = jnp.zeros((16,), jnp.float32)

    def init_step(t, _):
        code_s[t] = jnp.int32(-1)
        return 0
    lax.fori_loop(0, ROWS, init_step, 0)

    code_s[0] = jnp.int32(0)

    def samp_step(g, _):
        v = samp_v[pl.ds(g * 16, 16)]
        for i in range(16):
            code_s[v[i]] = g * 16 + i + 1
        return 0
    lax.fori_loop(0, N_VIS // 16, samp_step, 0)

    def mask_step(g, _):
        v = mask_v[pl.ds(g * 16, 16)]
        for i in range(16):
            code_s[v[i]] = jnp.int32(-2)
        return 0
    lax.fori_loop(0, N_MASK // 16, mask_step, 0)

    def fill_row(row, _):
        def fill_col(col, _):
            constbuf[row, 0, pl.ds(col * 16, 16)] = vs
            zerobuf[row, 0, pl.ds(col * 16, 16)] = vz
            return 0
        lax.fori_loop(0, EMBED // 16, fill_col, 0)
        return 0
    lax.fori_loop(0, HALF, fill_row, 0)

    def drain(sem):
        pltpu.make_async_copy(
            out_hbm.at[pl.ds(0, HALF), pl.ds(0, 1), :], constbuf, sem).wait()

    def fire_row(r, active, cw, cx):
        c = code_s[r]
        is_c = active & (c == -2)
        is_z = active & (c == -1)
        is_x = active & (c >= 0)
        dst = out_hbm.at[pl.ds(b0, HALF), pl.ds(r, 1), :]

        @pl.when(is_c)
        def _():
            pltpu.make_async_copy(constbuf, dst, sem_w).start()

        @pl.when(is_z)
        def _():
            pltpu.make_async_copy(zerobuf, dst, sem_w).start()

        cw = cw + jnp.where(is_c | is_z, 1, 0)

        @pl.when(cw > LAG)
        def _():
            drain(sem_w)
        cw = jnp.where(cw > LAG, cw - 1, cw)

        @pl.when(is_x & (cx > 0))
        def _():
            drain(sem_x)

        @pl.when(is_x)
        def _():
            pltpu.sync_copy(x_hbm.at[pl.ds(b0, HALF), pl.ds(c, 1), :], xbuf)
            pltpu.make_async_copy(xbuf, dst, sem_x).start()
        cx = jnp.where(is_x, 1, cx)
        return cw, cx

    def row_step(k, carry):
        cw, cx = carry
        return fire_row(k * NBH + rgrp, jnp.bool_(True), cw, cx)

    cw, cx = lax.fori_loop(0, NUM_PATCHES // NBH, row_step,
                           (jnp.int32(0), jnp.int32(0)))

    cw, cx = fire_row(NUM_PATCHES, rgrp == 0, cw, cx)

    for i in range(LAG + 1):
        @pl.when(cw > i)
        def _():
            drain(sem_w)

    @pl.when(cx > 0)
    def _():
        drain(sem_x)


@functools.partial(jax.jit, static_argnames=())
def kernel(x, sample_index, mask_index, W, b):
    w2d = W.reshape(EMBED, IN_CHANS * PATCH * PATCH)
    run = pl.kernel(
        _body,
        mesh=plsc.VectorSubcoreMesh(core_axis_name="c", subcore_axis_name="s"),
        out_type=jax.ShapeDtypeStruct((B, ROWS, EMBED), jnp.float32),
        scratch_types=[
            pltpu.VMEM((N_VIS,), jnp.int32),
            pltpu.VMEM((N_MASK,), jnp.int32),
            pltpu.SMEM((ROWS,), jnp.int32),
            pltpu.VMEM((1, EMBED), jnp.float32),
            pltpu.VMEM((16,), jnp.float32),
            pltpu.VMEM((HALF, 1, EMBED), jnp.float32),
            pltpu.VMEM((HALF, 1, EMBED), jnp.float32),
            pltpu.VMEM((HALF, 1, EMBED), jnp.float32),
            pltpu.SemaphoreType.DMA,
            pltpu.SemaphoreType.DMA,
        ],
    )
    return run(x, sample_index, mask_index, w2d, b)
